# baseline (speedup 1.0000x reference)
import functools

import jax
import jax.numpy as jnp
from jax import lax
from jax.experimental import pallas as pl
from jax.experimental.pallas import tpu as pltpu
from jax.experimental.pallas import tpu_sc as plsc

N = 10000
E = 320000
D = 128
H = 128

NC = 2
NS = 16
L = 16

HH = H // 2
NBUF = 4
C = 80
EW = E // NS
NCHUNK = EW // C
OB = C
NBLK = N // OB


def _gelu(x):
    return x * (0.5 * (1.0 + lax.erf(x * (2.0 ** -0.5))))


def _node_ffn(x, W1, c1, W2, c2):
    blk = 1000

    def body(x_ref, w1_ref, c1_ref, w2_ref, c2_ref, o_ref):
        h = jnp.dot(x_ref[...], w1_ref[...], preferred_element_type=jnp.float32)
        h = _gelu(h + c1_ref[...])
        h = jnp.dot(h, w2_ref[...], preferred_element_type=jnp.float32)
        h = _gelu(h + c2_ref[...])
        o_ref[0] = h[:, :HH]
        o_ref[1] = h[:, HH:]

    return pl.pallas_call(
        body,
        grid=(N // blk,),
        in_specs=[
            pl.BlockSpec((blk, D), lambda i: (i, 0)),
            pl.BlockSpec((D, H), lambda i: (0, 0)),
            pl.BlockSpec((1, H), lambda i: (0, 0)),
            pl.BlockSpec((H, H), lambda i: (0, 0)),
            pl.BlockSpec((1, H), lambda i: (0, 0)),
        ],
        out_specs=pl.BlockSpec((2, blk, HH), lambda i: (0, i, 0)),
        out_shape=jax.ShapeDtypeStruct((2, N, HH), jnp.float32),
    )(x, W1, c1, W2, c2)


def _bcast_lane(v16, lane):
    idx = jnp.full((L, 1), lane, dtype=jnp.int32)
    return lax.gather(
        v16, idx,
        lax.GatherDimensionNumbers(
            offset_dims=(), collapsed_slice_dims=(0,), start_index_map=(0,)),
        slice_sizes=(1,),
        mode=lax.GatherScatterMode.PROMISE_IN_BOUNDS)


def _sc_segment_sum(F2, nbr2d, dst2d, w):
    mesh = plsc.VectorSubcoreMesh(core_axis_name="c", subcore_axis_name="s")

    @functools.partial(
        pl.kernel,
        out_type=jax.ShapeDtypeStruct((NC, N, HH), jnp.float32),
        mesh=mesh,
        scratch_types=[
            pltpu.VMEM((NCHUNK, C), jnp.int32),
            pltpu.VMEM((NCHUNK, C), jnp.int32),
            pltpu.VMEM((EW,), jnp.float32),
            [pltpu.VMEM((C, HH), jnp.float32) for _ in range(NBUF)],
            pltpu.VMEM_SHARED((N, HH), jnp.float32),
            [pltpu.SemaphoreType.DMA for _ in range(NBUF)],
        ],
        compiler_params=pltpu.CompilerParams(use_tc_tiling_on_sc=False),
    )
    def k(f_hbm, nbr_hbm, dst_hbm, w_hbm, out_hbm,
          idx_v, dst_v, w_v, bufs, agg_sh, sems):
        stage_v = bufs[0]
        c = lax.axis_index("c")
        s = lax.axis_index("s")

        def zero_row(r, carry):
            for q in range(HH // L):
                stage_v[r, pl.ds(q * L, L)] = jnp.zeros((L,), jnp.float32)
            return carry
        lax.fori_loop(0, OB, zero_row, 0)
        for i in range(pl.cdiv(NBLK, NS)):
            b = s + NS * i

            @pl.when(b < NBLK)
            def _():
                r0 = pl.multiple_of(b * OB, 8)
                pltpu.sync_copy(stage_v, agg_sh.at[pl.ds(r0, OB)])

        row0 = s * NCHUNK
        pltpu.sync_copy(nbr_hbm.at[pl.ds(row0, NCHUNK)], idx_v)
        pltpu.sync_copy(dst_hbm.at[pl.ds(row0, NCHUNK)], dst_v)
        pltpu.sync_copy(w_hbm.at[pl.ds(s * EW, EW)], w_v)

        cN = c * N

        def shift_row(r, carry):
            for g in range(C // L):
                sl = pl.ds(g * L, L)
                idx_v[r, sl] = idx_v[r, sl] + cN
            return carry
        lax.fori_loop(0, NCHUNK, shift_row, 0)

        plsc.subcore_barrier()

        G, S = bufs[0:2], bufs[2:4]
        gs, ss = sems[0:2], sems[2:4]
        for b in range(2):
            pltpu.async_copy(f_hbm.at[idx_v.at[b]], G[b], gs[b])

        def outer(it, carry):
            j0 = it * 2
            for b in range(2):
                j = j0 + b
                pltpu.make_async_copy(f_hbm.at[idx_v.at[j]], G[b], gs[b]).wait()

                @pl.when(j >= 2)
                def _():
                    pltpu.make_async_copy(
                        S[b], agg_sh.at[dst_v.at[j - 2]], ss[b]).wait()

                def ggroup(g, carry):
                    w16 = w_v[pl.ds(j * C + g * L, L)]
                    e0 = g * L
                    for lane in range(L):
                        ws = _bcast_lane(w16, lane)
                        for q in range(HH // L):
                            sl = pl.ds(q * L, L)
                            S[b][e0 + lane, sl] = G[b][e0 + lane, sl] * ws
                    return carry
                lax.fori_loop(0, C // L, ggroup, 0)

                pltpu.async_copy(S[b], agg_sh.at[dst_v.at[j]], ss[b], add=True)

                @pl.when(j + 2 < NCHUNK)
                def _():
                    pltpu.async_copy(f_hbm.at[idx_v.at[j + 2]], G[b], gs[b])
            return carry
        lax.fori_loop(0, NCHUNK // 2, outer, 0)

        for b in range(2):
            pltpu.make_async_copy(
                S[b], agg_sh.at[dst_v.at[NCHUNK - 2 + b]], ss[b]).wait()

        plsc.subcore_barrier()

        for i in range(pl.cdiv(NBLK, NS)):
            b = s + NS * i

            @pl.when(b < NBLK)
            def _():
                r0 = pl.multiple_of(b * OB, 8)
                pltpu.sync_copy(agg_sh.at[pl.ds(r0, OB)], stage_v)
                pltpu.sync_copy(stage_v, out_hbm.at[c, pl.ds(r0, OB)])

    return k(F2, nbr2d, dst2d, w)


def _out_ffn(x, p, Wa, Wb, c1, W2, c2):
    blk = 1000

    def body(x_ref, p_ref, wa_ref, wb_ref, c1_ref, w2_ref, c2_ref, o_ref):
        h = jnp.dot(x_ref[...], wa_ref[...], preferred_element_type=jnp.float32)
        h = h + jnp.dot(p_ref[0], wb_ref[...][:HH], preferred_element_type=jnp.float32)
        h = h + jnp.dot(p_ref[1], wb_ref[...][HH:], preferred_element_type=jnp.float32)
        h = _gelu(h + c1_ref[...])
        h = jnp.dot(h, w2_ref[...], preferred_element_type=jnp.float32)
        h = _gelu(h + c2_ref[...])
        nrm = lax.rsqrt(jnp.maximum(jnp.sum(h * h, axis=-1, keepdims=True), 1e-12))
        o_ref[...] = h * nrm

    return pl.pallas_call(
        body,
        grid=(N // blk,),
        in_specs=[
            pl.BlockSpec((blk, D), lambda i: (i, 0)),
            pl.BlockSpec((NC, blk, HH), lambda i: (0, i, 0)),
            pl.BlockSpec((D, H), lambda i: (0, 0)),
            pl.BlockSpec((H, H), lambda i: (0, 0)),
            pl.BlockSpec((1, H), lambda i: (0, 0)),
            pl.BlockSpec((H, H), lambda i: (0, 0)),
            pl.BlockSpec((1, H), lambda i: (0, 0)),
        ],
        out_specs=pl.BlockSpec((blk, H), lambda i: (i, 0)),
        out_shape=jax.ShapeDtypeStruct((N, H), jnp.float32),
    )(x, p, Wa, Wb, c1, W2, c2)


def kernel(node_representations, branches, branch_weights,
           m_g1, m_b1, m_mu1, m_v1, m_W1, m_c1,
           m_g2, m_b2, m_mu2, m_v2, m_W2, m_c2,
           e_g1, e_b1, e_mu1, e_v1, e_W1, e_c1,
           e_g2, e_b2, e_mu2, e_v2, e_W2, e_c2):
    x = node_representations[0]
    dst = branches[0]
    nbr = branches[1]
    w = branch_weights[0, :, 0]

    s1 = m_g1 * lax.rsqrt(m_v1 + 1e-3)
    t1 = m_b1 - m_mu1 * s1
    mW1 = s1[:, None] * m_W1
    mc1 = (m_c1 + t1 @ m_W1)[None]
    s2 = m_g2 * lax.rsqrt(m_v2 + 1e-3)
    t2 = m_b2 - m_mu2 * s2
    mW2 = s2[:, None] * m_W2
    mc2 = (m_c2 + t2 @ m_W2)[None]

    F2 = _node_ffn(x, mW1, mc1, mW2, mc2).reshape(2 * N, HH)

    p = _sc_segment_sum(F2, nbr.reshape(E // C, C), dst.reshape(E // C, C), w)

    se = e_g1 * lax.rsqrt(e_v1 + 1e-3)
    te = e_b1 - e_mu1 * se
    eW1 = se[:, None] * e_W1
    ec1 = (e_c1 + te @ e_W1)[None]
    sf = e_g2 * lax.rsqrt(e_v2 + 1e-3)
    tf = e_b2 - e_mu2 * sf
    eW2 = sf[:, None] * e_W2
    ec2 = (e_c2 + tf @ e_W2)[None]

    out = _out_ffn(x, p, eW1[:D], eW1[D:], ec1, eW2, ec2)
    return out[None]


# speedup vs baseline: 1.0807x; 1.0807x over previous
import functools

import jax
import jax.numpy as jnp
from jax import lax
from jax.experimental import pallas as pl
from jax.experimental.pallas import tpu as pltpu
from jax.experimental.pallas import tpu_sc as plsc

N = 10000
E = 320000
D = 128
H = 128

NC = 2
NS = 16
L = 16

HH = H // 2
NBUF = 4
C = 80
EW = E // NS
NCHUNK = EW // C
OB = C
NBLK = N // OB


def _gelu(x):
    return x * (0.5 * (1.0 + lax.erf(x * (2.0 ** -0.5))))


def _bn(x, g_ref, b_ref, mu_ref, v_ref, lo=None, hi=None):
    sl = slice(lo, hi)
    s = g_ref[...][:, sl] * lax.rsqrt(v_ref[...][:, sl] + 1e-3)
    t = b_ref[...][:, sl] - mu_ref[...][:, sl] * s
    return x * s + t


def _node_ffn(x, g1, b1, mu1, v1, W1, c1, g2, b2, mu2, v2, W2, c2):
    blk = 1000

    def body(x_ref, g1_ref, b1_ref, mu1_ref, v1_ref, w1_ref, c1_ref,
             g2_ref, b2_ref, mu2_ref, v2_ref, w2_ref, c2_ref, o_ref):
        h = _bn(x_ref[...], g1_ref, b1_ref, mu1_ref, v1_ref)
        h = jnp.dot(h, w1_ref[...], preferred_element_type=jnp.float32)
        h = _gelu(h + c1_ref[...])
        h = _bn(h, g2_ref, b2_ref, mu2_ref, v2_ref)
        h = jnp.dot(h, w2_ref[...], preferred_element_type=jnp.float32)
        h = _gelu(h + c2_ref[...])
        o_ref[0] = h[:, :HH]
        o_ref[1] = h[:, HH:]

    vec = pl.BlockSpec((1, D), lambda i: (0, 0))
    mat = pl.BlockSpec((D, H), lambda i: (0, 0))
    return pl.pallas_call(
        body,
        grid=(N // blk,),
        in_specs=[pl.BlockSpec((blk, D), lambda i: (i, 0)),
                  vec, vec, vec, vec, mat, vec, vec, vec, vec, vec, mat, vec],
        out_specs=pl.BlockSpec((2, blk, HH), lambda i: (0, i, 0)),
        out_shape=jax.ShapeDtypeStruct((2, N, HH), jnp.float32),
    )(x, g1, b1, mu1, v1, W1, c1, g2, b2, mu2, v2, W2, c2)


def _bcast_lane(v16, lane):
    idx = jnp.full((L, 1), lane, dtype=jnp.int32)
    return lax.gather(
        v16, idx,
        lax.GatherDimensionNumbers(
            offset_dims=(), collapsed_slice_dims=(0,), start_index_map=(0,)),
        slice_sizes=(1,),
        mode=lax.GatherScatterMode.PROMISE_IN_BOUNDS)


def _sc_segment_sum(F2, br3d, w):
    mesh = plsc.VectorSubcoreMesh(core_axis_name="c", subcore_axis_name="s")

    @functools.partial(
        pl.kernel,
        out_type=jax.ShapeDtypeStruct((NC, N, HH), jnp.float32),
        mesh=mesh,
        scratch_types=[
            pltpu.VMEM((NCHUNK, C), jnp.int32),
            pltpu.VMEM((NCHUNK, C), jnp.int32),
            pltpu.VMEM((EW,), jnp.float32),
            [pltpu.VMEM((C, HH), jnp.float32) for _ in range(NBUF)],
            pltpu.VMEM_SHARED((N, HH), jnp.float32),
            [pltpu.SemaphoreType.DMA for _ in range(NBUF)],
        ],
        compiler_params=pltpu.CompilerParams(use_tc_tiling_on_sc=False),
    )
    def k(f_hbm, br_hbm, w_hbm, out_hbm,
          idx_v, dst_v, w_v, bufs, agg_sh, sems):
        stage_v = bufs[0]
        c = lax.axis_index("c")
        s = lax.axis_index("s")

        def zero_row(r, carry):
            for q in range(HH // L):
                stage_v[r, pl.ds(q * L, L)] = jnp.zeros((L,), jnp.float32)
            return carry
        lax.fori_loop(0, OB, zero_row, 0)
        for i in range(pl.cdiv(NBLK, NS)):
            b = s + NS * i

            @pl.when(b < NBLK)
            def _():
                r0 = pl.multiple_of(b * OB, 8)
                pltpu.sync_copy(stage_v, agg_sh.at[pl.ds(r0, OB)])

        row0 = s * NCHUNK
        pltpu.sync_copy(br_hbm.at[1, pl.ds(row0, NCHUNK)], idx_v)
        pltpu.sync_copy(br_hbm.at[0, pl.ds(row0, NCHUNK)], dst_v)
        pltpu.sync_copy(w_hbm.at[pl.ds(s * EW, EW)], w_v)

        cN = c * N

        def shift_row(r, carry):
            for g in range(C // L):
                sl = pl.ds(g * L, L)
                idx_v[r, sl] = idx_v[r, sl] + cN
            return carry
        lax.fori_loop(0, NCHUNK, shift_row, 0)

        plsc.subcore_barrier()

        G, S = bufs[0:2], bufs[2:4]
        gs, ss = sems[0:2], sems[2:4]
        for b in range(2):
            pltpu.async_copy(f_hbm.at[idx_v.at[b]], G[b], gs[b])

        def outer(it, carry):
            j0 = it * 2
            for b in range(2):
                j = j0 + b
                pltpu.make_async_copy(f_hbm.at[idx_v.at[j]], G[b], gs[b]).wait()

                @pl.when(j >= 2)
                def _():
                    pltpu.make_async_copy(
                        S[b], agg_sh.at[dst_v.at[j - 2]], ss[b]).wait()

                def ggroup(g, carry):
                    w16 = w_v[pl.ds(j * C + g * L, L)]
                    e0 = g * L
                    for lane in range(L):
                        ws = _bcast_lane(w16, lane)
                        for q in range(HH // L):
                            sl = pl.ds(q * L, L)
                            S[b][e0 + lane, sl] = G[b][e0 + lane, sl] * ws
                    return carry
                lax.fori_loop(0, C // L, ggroup, 0)

                pltpu.async_copy(S[b], agg_sh.at[dst_v.at[j]], ss[b], add=True)

                @pl.when(j + 2 < NCHUNK)
                def _():
                    pltpu.async_copy(f_hbm.at[idx_v.at[j + 2]], G[b], gs[b])
            return carry
        lax.fori_loop(0, NCHUNK // 2, outer, 0)

        for b in range(2):
            pltpu.make_async_copy(
                S[b], agg_sh.at[dst_v.at[NCHUNK - 2 + b]], ss[b]).wait()

        plsc.subcore_barrier()

        for i in range(pl.cdiv(NBLK, NS)):
            b = s + NS * i

            @pl.when(b < NBLK)
            def _():
                r0 = pl.multiple_of(b * OB, 8)
                pltpu.sync_copy(agg_sh.at[pl.ds(r0, OB)], stage_v)
                pltpu.sync_copy(stage_v, out_hbm.at[c, pl.ds(r0, OB)])

    return k(F2, br3d, w)


def _out_ffn(x, p, g1, b1, mu1, v1, W1, c1, g2, b2, mu2, v2, W2, c2):
    blk = 1000

    def body(x_ref, p_ref, g1_ref, b1_ref, mu1_ref, v1_ref, w1_ref, c1_ref,
             g2_ref, b2_ref, mu2_ref, v2_ref, w2_ref, c2_ref, o_ref):
        xb = _bn(x_ref[...], g1_ref, b1_ref, mu1_ref, v1_ref, 0, D)
        agg = jnp.concatenate([p_ref[0], p_ref[1]], axis=1)
        ab = _bn(agg, g1_ref, b1_ref, mu1_ref, v1_ref, D, D + H)
        w1 = w1_ref[...]
        h = jnp.dot(xb, w1[:D], preferred_element_type=jnp.float32)
        h = h + jnp.dot(ab, w1[D:], preferred_element_type=jnp.float32)
        h = _gelu(h + c1_ref[...])
        h = _bn(h, g2_ref, b2_ref, mu2_ref, v2_ref)
        h = jnp.dot(h, w2_ref[...], preferred_element_type=jnp.float32)
        h = _gelu(h + c2_ref[...])
        nrm = lax.rsqrt(jnp.maximum(jnp.sum(h * h, axis=-1, keepdims=True), 1e-12))
        o_ref[...] = h * nrm

    vec2 = pl.BlockSpec((1, D + H), lambda i: (0, 0))
    vec = pl.BlockSpec((1, H), lambda i: (0, 0))
    return pl.pallas_call(
        body,
        grid=(N // blk,),
        in_specs=[
            pl.BlockSpec((blk, D), lambda i: (i, 0)),
            pl.BlockSpec((NC, blk, HH), lambda i: (0, i, 0)),
            vec2, vec2, vec2, vec2,
            pl.BlockSpec((D + H, H), lambda i: (0, 0)), vec,
            vec, vec, vec, vec,
            pl.BlockSpec((H, H), lambda i: (0, 0)), vec,
        ],
        out_specs=pl.BlockSpec((blk, H), lambda i: (i, 0)),
        out_shape=jax.ShapeDtypeStruct((N, H), jnp.float32),
    )(x, p, g1, b1, mu1, v1, W1, c1, g2, b2, mu2, v2, W2, c2)


def kernel(node_representations, branches, branch_weights,
           m_g1, m_b1, m_mu1, m_v1, m_W1, m_c1,
           m_g2, m_b2, m_mu2, m_v2, m_W2, m_c2,
           e_g1, e_b1, e_mu1, e_v1, e_W1, e_c1,
           e_g2, e_b2, e_mu2, e_v2, e_W2, e_c2):
    x = node_representations[0]
    w = branch_weights.reshape(E)

    F2 = _node_ffn(x,
                   m_g1[None], m_b1[None], m_mu1[None], m_v1[None], m_W1,
                   m_c1[None],
                   m_g2[None], m_b2[None], m_mu2[None], m_v2[None], m_W2,
                   m_c2[None]).reshape(2 * N, HH)

    p = _sc_segment_sum(F2, branches.reshape(2, E // C, C), w)

    out = _out_ffn(x, p,
                   e_g1[None], e_b1[None], e_mu1[None], e_v1[None], e_W1,
                   e_c1[None],
                   e_g2[None], e_b2[None], e_mu2[None], e_v2[None], e_W2,
                   e_c2[None])
    return out[None]


# speedup vs baseline: 1.0850x; 1.0040x over previous
import functools

import jax
import jax.numpy as jnp
from jax import lax
from jax.experimental import pallas as pl
from jax.experimental.pallas import tpu as pltpu
from jax.experimental.pallas import tpu_sc as plsc

N = 10000
E = 320000
D = 128
H = 128

NC = 2
NS = 16
L = 16

HH = H // 2
NBUF = 4
C = 80
EW = E // NS
NCHUNK = EW // C
OB = C
NBLK = N // OB


def _gelu(x):
    return x * (0.5 * (1.0 + lax.erf(x * (2.0 ** -0.5))))


def _bn(x, g_ref, b_ref, mu_ref, v_ref, lo=None, hi=None):
    sl = slice(lo, hi)
    s = g_ref[...][:, sl] * lax.rsqrt(v_ref[...][:, sl] + 1e-3)
    t = b_ref[...][:, sl] - mu_ref[...][:, sl] * s
    return x * s + t


def _node_ffn(x, g1, b1, mu1, v1, W1, c1, g2, b2, mu2, v2, W2, c2):
    blk = 1000

    def body(x_ref, g1_ref, b1_ref, mu1_ref, v1_ref, w1_ref, c1_ref,
             g2_ref, b2_ref, mu2_ref, v2_ref, w2_ref, c2_ref, o_ref):
        h = _bn(x_ref[...], g1_ref, b1_ref, mu1_ref, v1_ref)
        h = jnp.dot(h, w1_ref[...], preferred_element_type=jnp.float32)
        h = _gelu(h + c1_ref[...])
        h = _bn(h, g2_ref, b2_ref, mu2_ref, v2_ref)
        h = jnp.dot(h, w2_ref[...], preferred_element_type=jnp.float32)
        h = _gelu(h + c2_ref[...])
        o_ref[0] = h[:, :HH]
        o_ref[1] = h[:, HH:]

    vec = pl.BlockSpec((1, D), lambda i: (0, 0))
    mat = pl.BlockSpec((D, H), lambda i: (0, 0))
    return pl.pallas_call(
        body,
        grid=(N // blk,),
        in_specs=[pl.BlockSpec((blk, D), lambda i: (i, 0)),
                  vec, vec, vec, vec, mat, vec, vec, vec, vec, vec, mat, vec],
        out_specs=pl.BlockSpec((2, blk, HH), lambda i: (0, i, 0)),
        out_shape=jax.ShapeDtypeStruct((2, N, HH), jnp.float32),
    )(x, g1, b1, mu1, v1, W1, c1, g2, b2, mu2, v2, W2, c2)


def _bcast_lane(v16, lane):
    idx = jnp.full((L, 1), lane, dtype=jnp.int32)
    return lax.gather(
        v16, idx,
        lax.GatherDimensionNumbers(
            offset_dims=(), collapsed_slice_dims=(0,), start_index_map=(0,)),
        slice_sizes=(1,),
        mode=lax.GatherScatterMode.PROMISE_IN_BOUNDS)


def _sc_segment_sum(F2, br3d, w):
    mesh = plsc.VectorSubcoreMesh(core_axis_name="c", subcore_axis_name="s")

    @functools.partial(
        pl.kernel,
        out_type=jax.ShapeDtypeStruct((NC, N, HH), jnp.float32),
        mesh=mesh,
        scratch_types=[
            pltpu.VMEM((NCHUNK, C), jnp.int32),
            pltpu.VMEM((NCHUNK, C), jnp.int32),
            pltpu.VMEM((EW,), jnp.float32),
            [pltpu.VMEM((C, HH), jnp.float32) for _ in range(NBUF)],
            pltpu.VMEM_SHARED((N, HH), jnp.float32),
            [pltpu.SemaphoreType.DMA for _ in range(NBUF)],
        ],
        compiler_params=pltpu.CompilerParams(use_tc_tiling_on_sc=False),
    )
    def k(f_hbm, br_hbm, w_hbm, out_hbm,
          idx_v, dst_v, w_v, bufs, agg_sh, sems):
        c = lax.axis_index("c")
        s = lax.axis_index("s")
        G, S = bufs[0:2], bufs[2:4]
        gs, ss = sems[0:2], sems[2:4]
        stage_v = bufs[3]

        row0 = s * NCHUNK
        pltpu.sync_copy(br_hbm.at[1, pl.ds(row0, NCHUNK)], idx_v)
        pltpu.sync_copy(br_hbm.at[0, pl.ds(row0, NCHUNK)], dst_v)
        pltpu.sync_copy(w_hbm.at[pl.ds(s * EW, EW)], w_v)

        cN = c * N

        def shift_row(r, carry):
            for g in range(C // L):
                sl = pl.ds(g * L, L)
                idx_v[r, sl] = idx_v[r, sl] + cN
            return carry
        lax.fori_loop(0, NCHUNK, shift_row, 0)

        for b in range(2):
            pltpu.async_copy(f_hbm.at[idx_v.at[b]], G[b], gs[b])

        def zero_row(r, carry):
            for q in range(HH // L):
                stage_v[r, pl.ds(q * L, L)] = jnp.zeros((L,), jnp.float32)
            return carry
        lax.fori_loop(0, OB, zero_row, 0)
        for i in range(pl.cdiv(NBLK, NS)):
            b = s + NS * i

            @pl.when(b < NBLK)
            def _():
                r0 = pl.multiple_of(b * OB, 8)
                pltpu.sync_copy(stage_v, agg_sh.at[pl.ds(r0, OB)])

        plsc.subcore_barrier()


        def outer(it, carry):
            j0 = it * 2
            for b in range(2):
                j = j0 + b
                pltpu.make_async_copy(f_hbm.at[idx_v.at[j]], G[b], gs[b]).wait()

                @pl.when(j >= 2)
                def _():
                    pltpu.make_async_copy(
                        S[b], agg_sh.at[dst_v.at[j - 2]], ss[b]).wait()

                def ggroup(g, carry):
                    w16 = w_v[pl.ds(j * C + g * L, L)]
                    e0 = g * L
                    for lane in range(L):
                        ws = _bcast_lane(w16, lane)
                        for q in range(HH // L):
                            sl = pl.ds(q * L, L)
                            S[b][e0 + lane, sl] = G[b][e0 + lane, sl] * ws
                    return carry
                lax.fori_loop(0, C // L, ggroup, 0)

                pltpu.async_copy(S[b], agg_sh.at[dst_v.at[j]], ss[b], add=True)

                @pl.when(j + 2 < NCHUNK)
                def _():
                    pltpu.async_copy(f_hbm.at[idx_v.at[j + 2]], G[b], gs[b])
            return carry
        lax.fori_loop(0, NCHUNK // 2, outer, 0)

        for b in range(2):
            pltpu.make_async_copy(
                S[b], agg_sh.at[dst_v.at[NCHUNK - 2 + b]], ss[b]).wait()

        plsc.subcore_barrier()

        for i in range(pl.cdiv(NBLK, NS)):
            b = s + NS * i

            @pl.when(b < NBLK)
            def _():
                r0 = pl.multiple_of(b * OB, 8)
                pltpu.sync_copy(agg_sh.at[pl.ds(r0, OB)], stage_v)
                pltpu.sync_copy(stage_v, out_hbm.at[c, pl.ds(r0, OB)])

    return k(F2, br3d, w)


def _out_ffn(x, p, g1, b1, mu1, v1, W1, c1, g2, b2, mu2, v2, W2, c2):
    blk = 1000

    def body(x_ref, p_ref, g1_ref, b1_ref, mu1_ref, v1_ref, w1_ref, c1_ref,
             g2_ref, b2_ref, mu2_ref, v2_ref, w2_ref, c2_ref, o_ref):
        xb = _bn(x_ref[...], g1_ref, b1_ref, mu1_ref, v1_ref, 0, D)
        agg = jnp.concatenate([p_ref[0], p_ref[1]], axis=1)
        ab = _bn(agg, g1_ref, b1_ref, mu1_ref, v1_ref, D, D + H)
        w1 = w1_ref[...]
        h = jnp.dot(xb, w1[:D], preferred_element_type=jnp.float32)
        h = h + jnp.dot(ab, w1[D:], preferred_element_type=jnp.float32)
        h = _gelu(h + c1_ref[...])
        h = _bn(h, g2_ref, b2_ref, mu2_ref, v2_ref)
        h = jnp.dot(h, w2_ref[...], preferred_element_type=jnp.float32)
        h = _gelu(h + c2_ref[...])
        nrm = lax.rsqrt(jnp.maximum(jnp.sum(h * h, axis=-1, keepdims=True), 1e-12))
        o_ref[...] = h * nrm

    vec2 = pl.BlockSpec((1, D + H), lambda i: (0, 0))
    vec = pl.BlockSpec((1, H), lambda i: (0, 0))
    return pl.pallas_call(
        body,
        grid=(N // blk,),
        in_specs=[
            pl.BlockSpec((blk, D), lambda i: (i, 0)),
            pl.BlockSpec((NC, blk, HH), lambda i: (0, i, 0)),
            vec2, vec2, vec2, vec2,
            pl.BlockSpec((D + H, H), lambda i: (0, 0)), vec,
            vec, vec, vec, vec,
            pl.BlockSpec((H, H), lambda i: (0, 0)), vec,
        ],
        out_specs=pl.BlockSpec((blk, H), lambda i: (i, 0)),
        out_shape=jax.ShapeDtypeStruct((N, H), jnp.float32),
    )(x, p, g1, b1, mu1, v1, W1, c1, g2, b2, mu2, v2, W2, c2)


def kernel(node_representations, branches, branch_weights,
           m_g1, m_b1, m_mu1, m_v1, m_W1, m_c1,
           m_g2, m_b2, m_mu2, m_v2, m_W2, m_c2,
           e_g1, e_b1, e_mu1, e_v1, e_W1, e_c1,
           e_g2, e_b2, e_mu2, e_v2, e_W2, e_c2):
    x = node_representations[0]
    w = branch_weights.reshape(E)

    F2 = _node_ffn(x,
                   m_g1[None], m_b1[None], m_mu1[None], m_v1[None], m_W1,
                   m_c1[None],
                   m_g2[None], m_b2[None], m_mu2[None], m_v2[None], m_W2,
                   m_c2[None]).reshape(2 * N, HH)

    p = _sc_segment_sum(F2, branches.reshape(2, E // C, C), w)

    out = _out_ffn(x, p,
                   e_g1[None], e_b1[None], e_mu1[None], e_v1[None], e_W1,
                   e_c1[None],
                   e_g2[None], e_b2[None], e_mu2[None], e_v2[None], e_W2,
                   e_c2[None])
    return out[None]


# speedup vs baseline: 1.0926x; 1.0070x over previous
import functools

import jax
import jax.numpy as jnp
from jax import lax
from jax.experimental import pallas as pl
from jax.experimental.pallas import tpu as pltpu
from jax.experimental.pallas import tpu_sc as plsc

N = 10000
E = 320000
D = 128
H = 128

NC = 2
NS = 16
L = 16

HH = H // 2
NBUF = 4
C = 80
EW = E // NS
NCHUNK = EW // C
OB = C
NBLK = N // OB


def _gelu(x):
    return x * (0.5 * (1.0 + lax.erf(x * (2.0 ** -0.5))))


def _bn(x, g_ref, b_ref, mu_ref, v_ref, lo=None, hi=None):
    sl = slice(lo, hi)
    s = g_ref[...][:, sl] * lax.rsqrt(v_ref[...][:, sl] + 1e-3)
    t = b_ref[...][:, sl] - mu_ref[...][:, sl] * s
    return x * s + t


def _node_ffn(x, g1, b1, mu1, v1, W1, c1, g2, b2, mu2, v2, W2, c2):
    blk = 1000

    def body(x_ref, g1_ref, b1_ref, mu1_ref, v1_ref, w1_ref, c1_ref,
             g2_ref, b2_ref, mu2_ref, v2_ref, w2_ref, c2_ref, o_ref):
        h = _bn(x_ref[...], g1_ref, b1_ref, mu1_ref, v1_ref)
        h = jnp.dot(h, w1_ref[...], preferred_element_type=jnp.float32)
        h = _gelu(h + c1_ref[...])
        h = _bn(h, g2_ref, b2_ref, mu2_ref, v2_ref)
        h = jnp.dot(h, w2_ref[...], preferred_element_type=jnp.float32)
        h = _gelu(h + c2_ref[...])
        o_ref[0] = h[:, :HH]
        o_ref[1] = h[:, HH:]

    vec = pl.BlockSpec((1, D), lambda i: (0, 0))
    mat = pl.BlockSpec((D, H), lambda i: (0, 0))
    return pl.pallas_call(
        body,
        grid=(N // blk,),
        in_specs=[pl.BlockSpec((blk, D), lambda i: (i, 0)),
                  vec, vec, vec, vec, mat, vec, vec, vec, vec, vec, mat, vec],
        out_specs=pl.BlockSpec((2, blk, HH), lambda i: (0, i, 0)),
        out_shape=jax.ShapeDtypeStruct((2, N, HH), jnp.float32),
    )(x, g1, b1, mu1, v1, W1, c1, g2, b2, mu2, v2, W2, c2)


def _bcast_lane(v16, lane):
    idx = jnp.full((L, 1), lane, dtype=jnp.int32)
    return lax.gather(
        v16, idx,
        lax.GatherDimensionNumbers(
            offset_dims=(), collapsed_slice_dims=(0,), start_index_map=(0,)),
        slice_sizes=(1,),
        mode=lax.GatherScatterMode.PROMISE_IN_BOUNDS)


def _sc_segment_sum(F2, br3d, w):
    mesh = plsc.VectorSubcoreMesh(core_axis_name="c", subcore_axis_name="s")

    @functools.partial(
        pl.kernel,
        out_type=jax.ShapeDtypeStruct((NC, N, HH), jnp.float32),
        mesh=mesh,
        scratch_types=[
            pltpu.VMEM((NCHUNK, C), jnp.int32),
            pltpu.VMEM((NCHUNK, C), jnp.int32),
            pltpu.VMEM((EW,), jnp.float32),
            [pltpu.VMEM((C, HH), jnp.float32) for _ in range(NBUF)],
            pltpu.VMEM_SHARED((N, HH), jnp.float32),
            [pltpu.SemaphoreType.DMA for _ in range(NBUF)],
        ],
        compiler_params=pltpu.CompilerParams(use_tc_tiling_on_sc=False),
    )
    def k(f_hbm, br_hbm, w_hbm, out_hbm,
          idx_v, dst_v, w_v, bufs, agg_sh, sems):
        c = lax.axis_index("c")
        s = lax.axis_index("s")
        G, S = bufs[0:2], bufs[2:4]
        gs, ss = sems[0:2], sems[2:4]
        stage_v = bufs[3]

        row0 = s * NCHUNK
        pltpu.sync_copy(br_hbm.at[1, pl.ds(row0, NCHUNK)], idx_v)
        pltpu.sync_copy(br_hbm.at[0, pl.ds(row0, NCHUNK)], dst_v)
        pltpu.sync_copy(w_hbm.at[pl.ds(s * EW, EW)], w_v)

        cN = c * N

        def shift_row(r, carry):
            for g in range(C // L):
                sl = pl.ds(g * L, L)
                idx_v[r, sl] = idx_v[r, sl] + cN
            return carry
        lax.fori_loop(0, NCHUNK, shift_row, 0)

        for b in range(2):
            pltpu.async_copy(f_hbm.at[idx_v.at[b]], G[b], gs[b])

        def zero_row(r, carry):
            for q in range(HH // L):
                stage_v[r, pl.ds(q * L, L)] = jnp.zeros((L,), jnp.float32)
            return carry
        lax.fori_loop(0, OB, zero_row, 0)
        for i in range(pl.cdiv(NBLK, NS)):
            b = s + NS * i

            @pl.when(b < NBLK)
            def _():
                r0 = pl.multiple_of(b * OB, 8)
                pltpu.sync_copy(stage_v, agg_sh.at[pl.ds(r0, OB)])

        plsc.subcore_barrier()


        def outer(it, carry):
            j0 = it * 2
            for b in range(2):
                j = j0 + b
                pltpu.make_async_copy(f_hbm.at[idx_v.at[j]], G[b], gs[b]).wait()

                @pl.when(j >= 2)
                def _():
                    pltpu.make_async_copy(
                        S[b], agg_sh.at[dst_v.at[j - 2]], ss[b]).wait()

                def ggroup(g, carry):
                    w16 = w_v[pl.ds(j * C + g * L, L)]
                    e0 = g * L
                    for lane in range(L):
                        ws = _bcast_lane(w16, lane)
                        for q in range(HH // L):
                            sl = pl.ds(q * L, L)
                            S[b][e0 + lane, sl] = G[b][e0 + lane, sl] * ws
                    return carry
                lax.fori_loop(0, C // L, ggroup, 0)

                pltpu.async_copy(S[b], agg_sh.at[dst_v.at[j]], ss[b], add=True)

                @pl.when(j + 2 < NCHUNK)
                def _():
                    pltpu.async_copy(f_hbm.at[idx_v.at[j + 2]], G[b], gs[b])
            return carry
        lax.fori_loop(0, NCHUNK // 2, outer, 0)

        for b in range(2):
            pltpu.make_async_copy(
                S[b], agg_sh.at[dst_v.at[NCHUNK - 2 + b]], ss[b]).wait()

        plsc.subcore_barrier()

        nro = pl.cdiv(NBLK, NS)
        for i in range(nro):
            b = s + NS * i

            @pl.when(b < NBLK)
            def _():
                st, sem = S[i % 2], ss[i % 2]
                if i >= 2:
                    rp = pl.multiple_of((b - 2 * NS) * OB, 8)
                    pltpu.make_async_copy(
                        st, out_hbm.at[c, pl.ds(rp, OB)], sem).wait()
                r0 = pl.multiple_of(b * OB, 8)
                pltpu.sync_copy(agg_sh.at[pl.ds(r0, OB)], st)
                pltpu.async_copy(st, out_hbm.at[c, pl.ds(r0, OB)], sem)
        for i in range(nro):
            b = s + NS * i

            @pl.when((b < NBLK) & (b + 2 * NS >= NBLK))
            def _():
                r0 = pl.multiple_of(b * OB, 8)
                pltpu.make_async_copy(
                    S[i % 2], out_hbm.at[c, pl.ds(r0, OB)], ss[i % 2]).wait()

    return k(F2, br3d, w)


def _out_ffn(x, p, g1, b1, mu1, v1, W1, c1, g2, b2, mu2, v2, W2, c2):
    blk = 1000

    def body(x_ref, p_ref, g1_ref, b1_ref, mu1_ref, v1_ref, w1_ref, c1_ref,
             g2_ref, b2_ref, mu2_ref, v2_ref, w2_ref, c2_ref, o_ref):
        xb = _bn(x_ref[...], g1_ref, b1_ref, mu1_ref, v1_ref, 0, D)
        agg = jnp.concatenate([p_ref[0], p_ref[1]], axis=1)
        ab = _bn(agg, g1_ref, b1_ref, mu1_ref, v1_ref, D, D + H)
        w1 = w1_ref[...]
        h = jnp.dot(xb, w1[:D], preferred_element_type=jnp.float32)
        h = h + jnp.dot(ab, w1[D:], preferred_element_type=jnp.float32)
        h = _gelu(h + c1_ref[...])
        h = _bn(h, g2_ref, b2_ref, mu2_ref, v2_ref)
        h = jnp.dot(h, w2_ref[...], preferred_element_type=jnp.float32)
        h = _gelu(h + c2_ref[...])
        nrm = lax.rsqrt(jnp.maximum(jnp.sum(h * h, axis=-1, keepdims=True), 1e-12))
        o_ref[...] = h * nrm

    vec2 = pl.BlockSpec((1, D + H), lambda i: (0, 0))
    vec = pl.BlockSpec((1, H), lambda i: (0, 0))
    return pl.pallas_call(
        body,
        grid=(N // blk,),
        in_specs=[
            pl.BlockSpec((blk, D), lambda i: (i, 0)),
            pl.BlockSpec((NC, blk, HH), lambda i: (0, i, 0)),
            vec2, vec2, vec2, vec2,
            pl.BlockSpec((D + H, H), lambda i: (0, 0)), vec,
            vec, vec, vec, vec,
            pl.BlockSpec((H, H), lambda i: (0, 0)), vec,
        ],
        out_specs=pl.BlockSpec((blk, H), lambda i: (i, 0)),
        out_shape=jax.ShapeDtypeStruct((N, H), jnp.float32),
    )(x, p, g1, b1, mu1, v1, W1, c1, g2, b2, mu2, v2, W2, c2)


def kernel(node_representations, branches, branch_weights,
           m_g1, m_b1, m_mu1, m_v1, m_W1, m_c1,
           m_g2, m_b2, m_mu2, m_v2, m_W2, m_c2,
           e_g1, e_b1, e_mu1, e_v1, e_W1, e_c1,
           e_g2, e_b2, e_mu2, e_v2, e_W2, e_c2):
    x = node_representations[0]
    w = branch_weights.reshape(E)

    F2 = _node_ffn(x,
                   m_g1[None], m_b1[None], m_mu1[None], m_v1[None], m_W1,
                   m_c1[None],
                   m_g2[None], m_b2[None], m_mu2[None], m_v2[None], m_W2,
                   m_c2[None]).reshape(2 * N, HH)

    p = _sc_segment_sum(F2, branches.reshape(2, E // C, C), w)

    out = _out_ffn(x, p,
                   e_g1[None], e_b1[None], e_mu1[None], e_v1[None], e_W1,
                   e_c1[None],
                   e_g2[None], e_b2[None], e_mu2[None], e_v2[None], e_W2,
                   e_c2[None])
    return out[None]
